# baseline (speedup 1.0000x reference)
import functools

import jax
import jax.numpy as jnp
from jax import lax
from jax.experimental import pallas as pl
from jax.experimental.pallas import tpu as pltpu
from jax.experimental.pallas import tpu_sc as plsc

B = 1024
KP1 = 256
D = 64
N = 100000
MOM = 0.5
NC = 2
NS = 16
NW = NC * NS
BPW = B // NW

_mesh = plsc.VectorSubcoreMesh(
    core_axis_name="c", subcore_axis_name="s", num_cores=NC, num_subcores=NS
)


def _f32(*s):
    return jax.ShapeDtypeStruct(s, jnp.float32)


@functools.partial(
    pl.kernel,
    out_type=tuple(_f32(B, KP1) for _ in range(6)) + (_f32(3, B, D),),
    mesh=_mesh,
    compiler_params=pltpu.CompilerParams(
        needs_layout_passes=False, use_tc_tiling_on_sc=False),
    scratch_types=[
        pltpu.VMEM((2, 128), jnp.int32),
        pltpu.VMEM((KP1, D), jnp.float32),
        pltpu.VMEM((KP1, D), jnp.float32),
        pltpu.VMEM((KP1, D), jnp.float32),
        pltpu.VMEM((BPW, D), jnp.float32),
        pltpu.VMEM((BPW, D), jnp.float32),
        pltpu.VMEM((BPW, D), jnp.float32),
        pltpu.VMEM((B,), jnp.int32),
        pltpu.VMEM((BPW,), jnp.int32),
        pltpu.VMEM((BPW + 16,), jnp.int32),
        pltpu.VMEM((BPW,), jnp.int32),
        pltpu.VMEM((6, KP1), jnp.float32),
        pltpu.VMEM((6, 16, 16), jnp.float32),
        pltpu.VMEM((16, 16), jnp.int32),
        pltpu.VMEM((3, BPW, D), jnp.float32),
        pltpu.VMEM((3, BPW, D), jnp.float32),
        pltpu.VMEM((3, BPW, D), jnp.float32),
        pltpu.SemaphoreType.DMA,
        pltpu.SemaphoreType.DMA,
    ],
)
def _nce_sc(l_h, ab_h, ori_h, y_h, idx_h, ml_h, mab_h, mori_h,
            o0, o1, o2, o3, o4, o5, upd_o,
            idx2, rows_l, rows_ab, rows_ori, vl, vab, vori,
            y_all, ys, ysp, lastj, out6, redbuf, ljbuf,
            posb, featb, updb, sem, sem2):
    c = lax.axis_index("c")
    s = lax.axis_index("s")
    w = s * NC + c
    b0 = w * BPW
    iota16 = lax.iota(jnp.int32, 16)

    pltpu.sync_copy(y_h, y_all)
    pltpu.sync_copy(y_h.at[pl.ds(b0, BPW)], ys)
    pltpu.sync_copy(y_h.at[pl.ds(b0, BPW)], ysp.at[pl.ds(0, BPW)])
    pltpu.sync_copy(l_h.at[pl.ds(b0, BPW)], vl)
    pltpu.sync_copy(ab_h.at[pl.ds(b0, BPW)], vab)
    pltpu.sync_copy(ori_h.at[pl.ds(b0, BPW)], vori)

    @pl.loop(0, BPW)
    def _batch(t):
        b = b0 + t
        pltpu.sync_copy(idx_h.at[b, pl.ds(0, 128)], idx2.at[0])
        pltpu.sync_copy(idx_h.at[b, pl.ds(128, 128)], idx2.at[1])
        yb = ysp[pl.ds(t, 16)][0]
        v0 = idx2[0, pl.ds(0, 16)]
        idx2[0, pl.ds(0, 16)] = jnp.where(iota16 == 0, yb, v0)

        hs = []
        for mh, rv in ((ml_h, rows_l), (mab_h, rows_ab), (mori_h, rows_ori)):
            for half in range(2):
                hs.append(pltpu.async_copy(
                    mh.at[idx2.at[half]], rv.at[pl.ds(half * 128, 128)], sem))
        for h in hs:
            h.wait()

        lvec = [vl[t, pl.ds(16 * j, 16)] for j in range(4)]
        avec = [vab[t, pl.ds(16 * j, 16)] for j in range(4)]
        ovec = [vori[t, pl.ds(16 * j, 16)] for j in range(4)]

        @pl.loop(0, KP1 // 16)
        def _kc(kc):
            base = kc * 16
            for j in range(16):
                k = base + j
                wlk = [rows_l[k, pl.ds(16 * q, 16)] for q in range(4)]
                wak = [rows_ab[k, pl.ds(16 * q, 16)] for q in range(4)]
                wok = [rows_ori[k, pl.ds(16 * q, 16)] for q in range(4)]

                def pdot(wv, vv):
                    return (wv[0] * vv[0] + wv[1] * vv[1]
                            + wv[2] * vv[2] + wv[3] * vv[3])

                colj = jnp.full((16,), j, jnp.int32)
                pds = (pdot(wlk, avec),
                       pdot(wak, lvec),
                       pdot(wlk, ovec),
                       pdot(wok, lvec),
                       pdot(wok, avec),
                       pdot(wak, ovec))
                for i in range(6):
                    plsc.store_scatter(redbuf.at[i], (iota16, colj), pds[i])
            for i in range(6):
                acc = redbuf[i, 0, pl.ds(0, 16)]
                for r in range(1, 16):
                    acc = acc + redbuf[i, r, pl.ds(0, 16)]
                out6[i, pl.ds(base, 16)] = acc

        ho = []
        for i, oref in enumerate((o0, o1, o2, o3, o4, o5)):
            ho.append(pltpu.async_copy(out6.at[i], oref.at[b], sem2))
        for h in ho:
            h.wait()

    for tc in range(BPW // 16):
        for j in range(16):
            t = tc * 16 + j
            yi = ysp[pl.ds(t, 16)][0]

            @pl.loop(0, B // 16, init_carry=jnp.full((16,), -1, jnp.int32))
            def best(ci, acc):
                yv = y_all[pl.ds(ci * 16, 16)]
                jv = iota16 + ci * 16
                return jnp.maximum(acc, jnp.where(yv == yi, jv, -1))

            plsc.store_scatter(ljbuf, (iota16, jnp.full((16,), j, jnp.int32)),
                               best)
        mx = ljbuf[0, pl.ds(0, 16)]
        for r in range(1, 16):
            mx = jnp.maximum(mx, ljbuf[r, pl.ds(0, 16)])
        lastj[pl.ds(tc * 16, 16)] = mx

    hs = []
    for i, fh in enumerate((l_h, ab_h, ori_h)):
        hs.append(pltpu.async_copy(fh.at[lastj], featb.at[i], sem))
    for i, mh in enumerate((ml_h, mab_h, mori_h)):
        hs.append(pltpu.async_copy(mh.at[ys], posb.at[i], sem))
    for h in hs:
        h.wait()

    def _pos(m, t):
        return [posb[m, t, pl.ds(16 * j, 16)] * MOM
                + featb[m, t, pl.ds(16 * j, 16)] * (1.0 - MOM)
                for j in range(4)]

    for m in range(3):
        for g in range(BPW // 16):
            for j in range(16):
                pv = _pos(m, g * 16 + j)
                sq = (pv[0] * pv[0] + pv[1] * pv[1]
                      + pv[2] * pv[2] + pv[3] * pv[3])
                plsc.store_scatter(redbuf.at[0],
                                   (iota16, jnp.full((16,), j, jnp.int32)), sq)
            ns = redbuf[0, 0, pl.ds(0, 16)]
            for r in range(1, 16):
                ns = ns + redbuf[0, r, pl.ds(0, 16)]
            bits = plsc.bitcast(ns, jnp.int32)
            bits = jnp.int32(0x5F3759DF) - (bits >> 1)
            r = plsc.bitcast(bits, jnp.float32)
            for _ in range(4):
                r = r * (1.5 - 0.5 * ns * r * r)
            for j in range(16):
                t = g * 16 + j
                pv = _pos(m, t)
                rj = r[j]
                for q in range(4):
                    updb[m, t, pl.ds(16 * q, 16)] = pv[q] * rj

    for m in range(3):
        pltpu.sync_copy(updb.at[m], upd_o.at[m, pl.ds(b0, BPW)])


RPW = N // NW


@functools.partial(
    pl.kernel,
    out_type=tuple(_f32(N + 8, D) for _ in range(3)),
    mesh=_mesh,
    compiler_params=pltpu.CompilerParams(
        needs_layout_passes=False, use_tc_tiling_on_sc=False),
    scratch_types=[
        pltpu.VMEM((8, 128), jnp.int32),
        pltpu.VMEM((128, D), jnp.float32),
        pltpu.VMEM((B,), jnp.int32),
        pltpu.SemaphoreType.DMA,
    ],
)
def _scatter_sc(y_h, upd_h, ml_h, mab_h, mori_h, nl_o, nab_o, nori_o,
                destidx, updchunk, yb, sem):
    c = lax.axis_index("c")
    s = lax.axis_index("s")
    w = s * NC + c
    r0 = w * RPW

    for mh, no in ((ml_h, nl_o), (mab_h, nab_o), (mori_h, nori_o)):
        pltpu.sync_copy(mh.at[pl.ds(r0, RPW)], no.at[pl.ds(r0, RPW)])

    pltpu.sync_copy(y_h, yb)
    for c8 in range(8):
        for j in range(8):
            yv = yb[pl.ds(c8 * 128 + j * 16, 16)]
            dv = jnp.where((yv >= r0) & (yv < r0 + RPW), yv, N)
            destidx[c8, pl.ds(j * 16, 16)] = dv

    for m, no in enumerate((nl_o, nab_o, nori_o)):
        for c8 in range(8):
            pltpu.sync_copy(upd_h.at[m, pl.ds(c8 * 128, 128)], updchunk)
            pltpu.async_copy(updchunk, no.at[destidx.at[c8]], sem).wait()


def kernel(l, ab, ori, y, idx, memory_l, memory_ab, memory_ori):
    y32 = y.astype(jnp.int32)
    idx32 = idx.astype(jnp.int32)
    o0, o1, o2, o3, o4, o5, upd = _nce_sc(
        l, ab, ori, y32, idx32, memory_l, memory_ab, memory_ori)
    nl, nab, nori = _scatter_sc(y32, upd, memory_l, memory_ab, memory_ori)
    return (o0[..., None], o1[..., None], o2[..., None], o3[..., None],
            o4[..., None], o5[..., None], nl[:N], nab[:N], nori[:N])


# speedup vs baseline: 1.0109x; 1.0109x over previous
import functools

import jax
import jax.numpy as jnp
from jax import lax
from jax.experimental import pallas as pl
from jax.experimental.pallas import tpu as pltpu
from jax.experimental.pallas import tpu_sc as plsc

B = 1024
KP1 = 256
D = 64
N = 100000
MOM = 0.5
NC = 2
NS = 16
NW = NC * NS
BPW = B // NW

_mesh = plsc.VectorSubcoreMesh(
    core_axis_name="c", subcore_axis_name="s", num_cores=NC, num_subcores=NS
)


def _f32(*s):
    return jax.ShapeDtypeStruct(s, jnp.float32)


@functools.partial(
    pl.kernel,
    out_type=tuple(_f32(B, KP1) for _ in range(6)) + (_f32(3, B, D),),
    mesh=_mesh,
    compiler_params=pltpu.CompilerParams(
        needs_layout_passes=False, use_tc_tiling_on_sc=False),
    scratch_types=[
        pltpu.VMEM((2, 128), jnp.int32),
        pltpu.VMEM((KP1, D), jnp.float32),
        pltpu.VMEM((KP1, D), jnp.float32),
        pltpu.VMEM((KP1, D), jnp.float32),
        pltpu.VMEM((BPW, D), jnp.float32),
        pltpu.VMEM((BPW, D), jnp.float32),
        pltpu.VMEM((BPW, D), jnp.float32),
        pltpu.VMEM((B,), jnp.int32),
        pltpu.VMEM((BPW,), jnp.int32),
        pltpu.VMEM((BPW + 16,), jnp.int32),
        pltpu.VMEM((BPW,), jnp.int32),
        pltpu.VMEM((6, KP1), jnp.float32),
        pltpu.VMEM((6, 16, 16), jnp.float32),
        pltpu.VMEM((16, 16), jnp.int32),
        pltpu.VMEM((3, BPW, D), jnp.float32),
        pltpu.VMEM((3, BPW, D), jnp.float32),
        pltpu.VMEM((3, BPW, D), jnp.float32),
        pltpu.SemaphoreType.DMA,
        pltpu.SemaphoreType.DMA,
    ],
)
def _nce_sc(l_h, ab_h, ori_h, y_h, idx_h, ml_h, mab_h, mori_h,
            o0, o1, o2, o3, o4, o5, upd_o,
            idx2, rows_l, rows_ab, rows_ori, vl, vab, vori,
            y_all, ys, ysp, lastj, out6, redbuf, ljbuf,
            posb, featb, updb, sem, sem2):
    c = lax.axis_index("c")
    s = lax.axis_index("s")
    w = s * NC + c
    b0 = w * BPW
    iota16 = lax.iota(jnp.int32, 16)

    pltpu.sync_copy(y_h, y_all)
    pltpu.sync_copy(y_h.at[pl.ds(b0, BPW)], ys)
    pltpu.sync_copy(y_h.at[pl.ds(b0, BPW)], ysp.at[pl.ds(0, BPW)])
    pltpu.sync_copy(l_h.at[pl.ds(b0, BPW)], vl)
    pltpu.sync_copy(ab_h.at[pl.ds(b0, BPW)], vab)
    pltpu.sync_copy(ori_h.at[pl.ds(b0, BPW)], vori)

    @pl.loop(0, BPW)
    def _batch(t):
        b = b0 + t
        pltpu.sync_copy(idx_h.at[b, pl.ds(0, 128)], idx2.at[0])
        pltpu.sync_copy(idx_h.at[b, pl.ds(128, 128)], idx2.at[1])
        yb = ysp[pl.ds(t, 16)][0]
        v0 = idx2[0, pl.ds(0, 16)]
        idx2[0, pl.ds(0, 16)] = jnp.where(iota16 == 0, yb, v0)

        hs = []
        for mh, rv in ((ml_h, rows_l), (mab_h, rows_ab), (mori_h, rows_ori)):
            for half in range(2):
                hs.append(pltpu.async_copy(
                    mh.at[idx2.at[half]], rv.at[pl.ds(half * 128, 128)], sem))
        for h in hs:
            h.wait()

        lvec = [vl[t, pl.ds(16 * j, 16)] for j in range(4)]
        avec = [vab[t, pl.ds(16 * j, 16)] for j in range(4)]
        ovec = [vori[t, pl.ds(16 * j, 16)] for j in range(4)]

        @pl.loop(0, KP1 // 16)
        def _kc(kc):
            base = kc * 16
            for j in range(16):
                k = base + j
                wlk = [rows_l[k, pl.ds(16 * q, 16)] for q in range(4)]
                wak = [rows_ab[k, pl.ds(16 * q, 16)] for q in range(4)]
                wok = [rows_ori[k, pl.ds(16 * q, 16)] for q in range(4)]

                def pdot(wv, vv):
                    return (wv[0] * vv[0] + wv[1] * vv[1]
                            + wv[2] * vv[2] + wv[3] * vv[3])

                colj = jnp.full((16,), j, jnp.int32)
                pds = (pdot(wlk, avec),
                       pdot(wak, lvec),
                       pdot(wlk, ovec),
                       pdot(wok, lvec),
                       pdot(wok, avec),
                       pdot(wak, ovec))
                for i in range(6):
                    plsc.store_scatter(redbuf.at[i], (iota16, colj), pds[i])
            for i in range(6):
                acc = redbuf[i, 0, pl.ds(0, 16)]
                for r in range(1, 16):
                    acc = acc + redbuf[i, r, pl.ds(0, 16)]
                out6[i, pl.ds(base, 16)] = acc

        ho = []
        for i, oref in enumerate((o0, o1, o2, o3, o4, o5)):
            ho.append(pltpu.async_copy(out6.at[i], oref.at[b], sem2))
        for h in ho:
            h.wait()

    for tc in range(BPW // 16):
        for j in range(16):
            t = tc * 16 + j
            yi = ysp[pl.ds(t, 16)][0]

            @pl.loop(0, B // 16, init_carry=jnp.full((16,), -1, jnp.int32))
            def best(ci, acc):
                yv = y_all[pl.ds(ci * 16, 16)]
                jv = iota16 + ci * 16
                return jnp.maximum(acc, jnp.where(yv == yi, jv, -1))

            plsc.store_scatter(ljbuf, (iota16, jnp.full((16,), j, jnp.int32)),
                               best)
        mx = ljbuf[0, pl.ds(0, 16)]
        for r in range(1, 16):
            mx = jnp.maximum(mx, ljbuf[r, pl.ds(0, 16)])
        lastj[pl.ds(tc * 16, 16)] = mx

    hs = []
    for i, fh in enumerate((l_h, ab_h, ori_h)):
        hs.append(pltpu.async_copy(fh.at[lastj], featb.at[i], sem))
    for i, mh in enumerate((ml_h, mab_h, mori_h)):
        hs.append(pltpu.async_copy(mh.at[ys], posb.at[i], sem))
    for h in hs:
        h.wait()

    def _pos(m, t):
        return [posb[m, t, pl.ds(16 * j, 16)] * MOM
                + featb[m, t, pl.ds(16 * j, 16)] * (1.0 - MOM)
                for j in range(4)]

    for m in range(3):
        for g in range(BPW // 16):
            for j in range(16):
                pv = _pos(m, g * 16 + j)
                sq = (pv[0] * pv[0] + pv[1] * pv[1]
                      + pv[2] * pv[2] + pv[3] * pv[3])
                plsc.store_scatter(redbuf.at[0],
                                   (iota16, jnp.full((16,), j, jnp.int32)), sq)
            ns = redbuf[0, 0, pl.ds(0, 16)]
            for r in range(1, 16):
                ns = ns + redbuf[0, r, pl.ds(0, 16)]
            bits = plsc.bitcast(ns, jnp.int32)
            bits = jnp.int32(0x5F3759DF) - (bits >> 1)
            r = plsc.bitcast(bits, jnp.float32)
            for _ in range(4):
                r = r * (1.5 - 0.5 * ns * r * r)
            for j in range(16):
                t = g * 16 + j
                pv = _pos(m, t)
                rj = r[j]
                for q in range(4):
                    updb[m, t, pl.ds(16 * q, 16)] = pv[q] * rj

    for m in range(3):
        pltpu.sync_copy(updb.at[m], upd_o.at[m, pl.ds(b0, BPW)])


RPW = N // NW


@functools.partial(
    pl.kernel,
    out_type=tuple(_f32(N + B, D) for _ in range(3)),
    mesh=_mesh,
    compiler_params=pltpu.CompilerParams(
        needs_layout_passes=False, use_tc_tiling_on_sc=False),
    scratch_types=[
        pltpu.VMEM((8, 128), jnp.int32),
        pltpu.VMEM((128, D), jnp.float32),
        pltpu.VMEM((B,), jnp.int32),
        pltpu.SemaphoreType.DMA,
    ],
)
def _scatter_sc(y_h, upd_h, ml_h, mab_h, mori_h, nl_o, nab_o, nori_o,
                destidx, updchunk, yb, sem):
    c = lax.axis_index("c")
    s = lax.axis_index("s")
    w = s * NC + c
    r0 = w * RPW
    iota16 = lax.iota(jnp.int32, 16)

    for mh, no in ((ml_h, nl_o), (mab_h, nab_o), (mori_h, nori_o)):
        pltpu.sync_copy(mh.at[pl.ds(r0, RPW)], no.at[pl.ds(r0, RPW)])

    pltpu.sync_copy(y_h, yb)
    for c8 in range(8):
        for j in range(8):
            i0 = c8 * 128 + j * 16
            yv = yb[pl.ds(i0, 16)]
            dv = jnp.where((yv >= r0) & (yv < r0 + RPW), yv, N + i0 + iota16)
            destidx[c8, pl.ds(j * 16, 16)] = dv

    for m, no in enumerate((nl_o, nab_o, nori_o)):
        for c8 in range(8):
            pltpu.sync_copy(upd_h.at[m, pl.ds(c8 * 128, 128)], updchunk)
            pltpu.async_copy(updchunk, no.at[destidx.at[c8]], sem).wait()


def kernel(l, ab, ori, y, idx, memory_l, memory_ab, memory_ori):
    y32 = y.astype(jnp.int32)
    idx32 = idx.astype(jnp.int32)
    o0, o1, o2, o3, o4, o5, upd = _nce_sc(
        l, ab, ori, y32, idx32, memory_l, memory_ab, memory_ori)
    nl, nab, nori = _scatter_sc(y32, upd, memory_l, memory_ab, memory_ori)
    return (o0[..., None], o1[..., None], o2[..., None], o3[..., None],
            o4[..., None], o5[..., None], nl[:N], nab[:N], nori[:N])


# speedup vs baseline: 3.0928x; 3.0595x over previous
import functools

import jax
import jax.numpy as jnp
from jax import lax
from jax.experimental import pallas as pl
from jax.experimental.pallas import tpu as pltpu
from jax.experimental.pallas import tpu_sc as plsc

B = 1024
KP1 = 256
D = 64
N = 100000
MOM = 0.5
NC = 2
NS = 16
NW = NC * NS
BPW = B // NW

_mesh = plsc.VectorSubcoreMesh(
    core_axis_name="c", subcore_axis_name="s", num_cores=NC, num_subcores=NS
)


def _f32(*s):
    return jax.ShapeDtypeStruct(s, jnp.float32)


@functools.partial(
    pl.kernel,
    out_type=tuple(_f32(B, KP1) for _ in range(6)) + (_f32(3, B, D),),
    mesh=_mesh,
    compiler_params=pltpu.CompilerParams(
        needs_layout_passes=False, use_tc_tiling_on_sc=False),
    scratch_types=[
        pltpu.VMEM((2, 128), jnp.int32),
        pltpu.VMEM((KP1, D), jnp.float32),
        pltpu.VMEM((KP1, D), jnp.float32),
        pltpu.VMEM((KP1, D), jnp.float32),
        pltpu.VMEM((BPW, D), jnp.float32),
        pltpu.VMEM((BPW, D), jnp.float32),
        pltpu.VMEM((BPW, D), jnp.float32),
        pltpu.VMEM((B,), jnp.int32),
        pltpu.VMEM((BPW,), jnp.int32),
        pltpu.VMEM((BPW + 16,), jnp.int32),
        pltpu.VMEM((BPW,), jnp.int32),
        pltpu.VMEM((6, KP1), jnp.float32),
        pltpu.VMEM((6, 16, 16), jnp.float32),
        pltpu.VMEM((16, 16), jnp.int32),
        pltpu.VMEM((3, BPW, D), jnp.float32),
        pltpu.VMEM((3, BPW, D), jnp.float32),
        pltpu.VMEM((3, BPW, D), jnp.float32),
        pltpu.SemaphoreType.DMA,
        pltpu.SemaphoreType.DMA,
    ],
)
def _nce_sc(l_h, ab_h, ori_h, y_h, idx_h, ml_h, mab_h, mori_h,
            o0, o1, o2, o3, o4, o5, upd_o,
            idx2, rows_l, rows_ab, rows_ori, vl, vab, vori,
            y_all, ys, ysp, lastj, out6, redbuf, ljbuf,
            posb, featb, updb, sem, sem2):
    c = lax.axis_index("c")
    s = lax.axis_index("s")
    w = s * NC + c
    b0 = w * BPW
    iota16 = lax.iota(jnp.int32, 16)

    pltpu.sync_copy(y_h, y_all)
    pltpu.sync_copy(y_h.at[pl.ds(b0, BPW)], ys)
    pltpu.sync_copy(y_h.at[pl.ds(b0, BPW)], ysp.at[pl.ds(0, BPW)])
    pltpu.sync_copy(l_h.at[pl.ds(b0, BPW)], vl)
    pltpu.sync_copy(ab_h.at[pl.ds(b0, BPW)], vab)
    pltpu.sync_copy(ori_h.at[pl.ds(b0, BPW)], vori)

    @pl.loop(0, BPW)
    def _batch(t):
        b = b0 + t
        pltpu.sync_copy(idx_h.at[b, pl.ds(0, 128)], idx2.at[0])
        pltpu.sync_copy(idx_h.at[b, pl.ds(128, 128)], idx2.at[1])
        yb = ysp[pl.ds(t, 16)][0]
        v0 = idx2[0, pl.ds(0, 16)]
        idx2[0, pl.ds(0, 16)] = jnp.where(iota16 == 0, yb, v0)

        hs = []
        for mh, rv in ((ml_h, rows_l), (mab_h, rows_ab), (mori_h, rows_ori)):
            for half in range(2):
                hs.append(pltpu.async_copy(
                    mh.at[idx2.at[half]], rv.at[pl.ds(half * 128, 128)], sem))
        for h in hs:
            h.wait()

        lvec = [vl[t, pl.ds(16 * j, 16)] for j in range(4)]
        avec = [vab[t, pl.ds(16 * j, 16)] for j in range(4)]
        ovec = [vori[t, pl.ds(16 * j, 16)] for j in range(4)]

        @pl.loop(0, KP1 // 16)
        def _kc(kc):
            base = kc * 16
            for j in range(16):
                k = base + j
                wlk = [rows_l[k, pl.ds(16 * q, 16)] for q in range(4)]
                wak = [rows_ab[k, pl.ds(16 * q, 16)] for q in range(4)]
                wok = [rows_ori[k, pl.ds(16 * q, 16)] for q in range(4)]

                def pdot(wv, vv):
                    return (wv[0] * vv[0] + wv[1] * vv[1]
                            + wv[2] * vv[2] + wv[3] * vv[3])

                colj = jnp.full((16,), j, jnp.int32)
                pds = (pdot(wlk, avec),
                       pdot(wak, lvec),
                       pdot(wlk, ovec),
                       pdot(wok, lvec),
                       pdot(wok, avec),
                       pdot(wak, ovec))
                for i in range(6):
                    plsc.store_scatter(redbuf.at[i], (iota16, colj), pds[i])
            for i in range(6):
                acc = redbuf[i, 0, pl.ds(0, 16)]
                for r in range(1, 16):
                    acc = acc + redbuf[i, r, pl.ds(0, 16)]
                out6[i, pl.ds(base, 16)] = acc

        ho = []
        for i, oref in enumerate((o0, o1, o2, o3, o4, o5)):
            ho.append(pltpu.async_copy(out6.at[i], oref.at[b], sem2))
        for h in ho:
            h.wait()

    for tc in range(BPW // 16):
        for j in range(16):
            t = tc * 16 + j
            yi = ysp[pl.ds(t, 16)][0]

            @pl.loop(0, B // 16, init_carry=jnp.full((16,), -1, jnp.int32))
            def best(ci, acc):
                yv = y_all[pl.ds(ci * 16, 16)]
                jv = iota16 + ci * 16
                return jnp.maximum(acc, jnp.where(yv == yi, jv, -1))

            plsc.store_scatter(ljbuf, (iota16, jnp.full((16,), j, jnp.int32)),
                               best)
        mx = ljbuf[0, pl.ds(0, 16)]
        for r in range(1, 16):
            mx = jnp.maximum(mx, ljbuf[r, pl.ds(0, 16)])
        lastj[pl.ds(tc * 16, 16)] = mx

    hs = []
    for i, fh in enumerate((l_h, ab_h, ori_h)):
        hs.append(pltpu.async_copy(fh.at[lastj], featb.at[i], sem))
    for i, mh in enumerate((ml_h, mab_h, mori_h)):
        hs.append(pltpu.async_copy(mh.at[ys], posb.at[i], sem))
    for h in hs:
        h.wait()

    def _pos(m, t):
        return [posb[m, t, pl.ds(16 * j, 16)] * MOM
                + featb[m, t, pl.ds(16 * j, 16)] * (1.0 - MOM)
                for j in range(4)]

    for m in range(3):
        for g in range(BPW // 16):
            for j in range(16):
                pv = _pos(m, g * 16 + j)
                sq = (pv[0] * pv[0] + pv[1] * pv[1]
                      + pv[2] * pv[2] + pv[3] * pv[3])
                plsc.store_scatter(redbuf.at[0],
                                   (iota16, jnp.full((16,), j, jnp.int32)), sq)
            ns = redbuf[0, 0, pl.ds(0, 16)]
            for r in range(1, 16):
                ns = ns + redbuf[0, r, pl.ds(0, 16)]
            bits = plsc.bitcast(ns, jnp.int32)
            bits = jnp.int32(0x5F3759DF) - (bits >> 1)
            r = plsc.bitcast(bits, jnp.float32)
            for _ in range(4):
                r = r * (1.5 - 0.5 * ns * r * r)
            for j in range(16):
                t = g * 16 + j
                pv = _pos(m, t)
                rj = r[j]
                for q in range(4):
                    updb[m, t, pl.ds(16 * q, 16)] = pv[q] * rj

    for m in range(3):
        pltpu.sync_copy(updb.at[m], upd_o.at[m, pl.ds(b0, BPW)])


RPW = N // NW


@functools.partial(
    pl.kernel,
    out_type=tuple(_f32(N + B, D) for _ in range(3)),
    mesh=_mesh,
    compiler_params=pltpu.CompilerParams(
        needs_layout_passes=False, use_tc_tiling_on_sc=False),
    scratch_types=[
        pltpu.VMEM((8, 128), jnp.int32),
        pltpu.VMEM((128, D), jnp.float32),
        pltpu.VMEM((B,), jnp.int32),
        pltpu.VMEM((2, 625, D), jnp.float32),
        pltpu.SemaphoreType.DMA,
        pltpu.SemaphoreType.DMA,
        pltpu.SemaphoreType.DMA,
    ],
)
def _scatter_sc(y_h, upd_h, ml_h, mab_h, mori_h, nl_o, nab_o, nori_o,
                destidx, updchunk, yb, copybuf, sem, semr, semw):
    c = lax.axis_index("c")
    s = lax.axis_index("s")
    w = s * NC + c
    r0 = w * RPW
    iota16 = lax.iota(jnp.int32, 16)

    NCH = RPW // 625
    hw = {}
    for mi, (mh, no) in enumerate(
            ((ml_h, nl_o), (mab_h, nab_o), (mori_h, nori_o))):
        for ci in range(NCH):
            gi = mi * NCH + ci
            p = gi % 2
            if gi >= 2:
                hw[gi - 2].wait()
            off = r0 + ci * 625
            pltpu.async_copy(mh.at[pl.ds(off, 625)], copybuf.at[p],
                             semr).wait()
            hw[gi] = pltpu.async_copy(copybuf.at[p], no.at[pl.ds(off, 625)],
                                      semw)
    hw[3 * NCH - 2].wait()
    hw[3 * NCH - 1].wait()

    pltpu.sync_copy(y_h, yb)
    for c8 in range(8):
        for j in range(8):
            i0 = c8 * 128 + j * 16
            yv = yb[pl.ds(i0, 16)]
            dv = jnp.where((yv >= r0) & (yv < r0 + RPW), yv, N + i0 + iota16)
            destidx[c8, pl.ds(j * 16, 16)] = dv

    for m, no in enumerate((nl_o, nab_o, nori_o)):
        for c8 in range(8):
            pltpu.sync_copy(upd_h.at[m, pl.ds(c8 * 128, 128)], updchunk)
            pltpu.async_copy(updchunk, no.at[destidx.at[c8]], sem).wait()


def kernel(l, ab, ori, y, idx, memory_l, memory_ab, memory_ori):
    y32 = y.astype(jnp.int32)
    idx32 = idx.astype(jnp.int32)
    o0, o1, o2, o3, o4, o5, upd = _nce_sc(
        l, ab, ori, y32, idx32, memory_l, memory_ab, memory_ori)
    nl, nab, nori = _scatter_sc(y32, upd, memory_l, memory_ab, memory_ori)
    return (o0[..., None], o1[..., None], o2[..., None], o3[..., None],
            o4[..., None], o5[..., None], nl[:N], nab[:N], nori[:N])


# speedup vs baseline: 3.3725x; 1.0904x over previous
import functools

import jax
import jax.numpy as jnp
from jax import lax
from jax.experimental import pallas as pl
from jax.experimental.pallas import tpu as pltpu
from jax.experimental.pallas import tpu_sc as plsc

B = 1024
KP1 = 256
D = 64
N = 100000
MOM = 0.5
NC = 2
NS = 16
NW = NC * NS
BPW = B // NW

_mesh = plsc.VectorSubcoreMesh(
    core_axis_name="c", subcore_axis_name="s", num_cores=NC, num_subcores=NS
)


def _f32(*s):
    return jax.ShapeDtypeStruct(s, jnp.float32)


@functools.partial(
    pl.kernel,
    out_type=tuple(_f32(B, KP1) for _ in range(6)) + (_f32(3, B, D),),
    mesh=_mesh,
    compiler_params=pltpu.CompilerParams(
        needs_layout_passes=False, use_tc_tiling_on_sc=False),
    scratch_types=[
        pltpu.VMEM((2, BPW, 128), jnp.int32),
        pltpu.VMEM((2, KP1, D), jnp.float32),
        pltpu.VMEM((2, KP1, D), jnp.float32),
        pltpu.VMEM((2, KP1, D), jnp.float32),
        pltpu.VMEM((BPW, D), jnp.float32),
        pltpu.VMEM((BPW, D), jnp.float32),
        pltpu.VMEM((BPW, D), jnp.float32),
        pltpu.VMEM((B,), jnp.int32),
        pltpu.VMEM((BPW,), jnp.int32),
        pltpu.VMEM((BPW + 16,), jnp.int32),
        pltpu.VMEM((BPW,), jnp.int32),
        pltpu.VMEM((6, KP1), jnp.float32),
        pltpu.VMEM((6, 16, 16), jnp.float32),
        pltpu.VMEM((16, 16), jnp.int32),
        pltpu.SemaphoreType.DMA,
        pltpu.SemaphoreType.DMA,
    ],
)
def _nce_sc(l_h, ab_h, ori_h, y_h, idx_h, ml_h, mab_h, mori_h,
            o0, o1, o2, o3, o4, o5, upd_o,
            idxall, rows_l, rows_ab, rows_ori, vl, vab, vori,
            y_all, ys, ysp, lastj, out6, redbuf, ljbuf,
            sem, sem2):
    c = lax.axis_index("c")
    s = lax.axis_index("s")
    w = s * NC + c
    b0 = w * BPW
    iota16 = lax.iota(jnp.int32, 16)

    pltpu.sync_copy(y_h, y_all)
    pltpu.sync_copy(y_h.at[pl.ds(b0, BPW)], ys)
    pltpu.sync_copy(y_h.at[pl.ds(b0, BPW)], ysp.at[pl.ds(0, BPW)])
    pltpu.sync_copy(l_h.at[pl.ds(b0, BPW)], vl)
    pltpu.sync_copy(ab_h.at[pl.ds(b0, BPW)], vab)
    pltpu.sync_copy(ori_h.at[pl.ds(b0, BPW)], vori)
    pltpu.sync_copy(idx_h.at[pl.ds(b0, BPW), pl.ds(0, 128)], idxall.at[0])
    pltpu.sync_copy(idx_h.at[pl.ds(b0, BPW), pl.ds(128, 128)], idxall.at[1])
    for t in range(BPW):
        yb = ysp[pl.ds(t, 16)][0]
        v0 = idxall[0, t, pl.ds(0, 16)]
        idxall[0, t, pl.ds(0, 16)] = jnp.where(iota16 == 0, yb, v0)

    def start_gathers(t, p):
        for mh, rv in ((ml_h, rows_l), (mab_h, rows_ab), (mori_h, rows_ori)):
            for half in range(2):
                pltpu.async_copy(
                    mh.at[idxall.at[half, t]],
                    rv.at[p].at[pl.ds(half * 128, 128)], sem)

    def drain_gathers(t, p):
        for mh, rv in ((ml_h, rows_l), (mab_h, rows_ab), (mori_h, rows_ori)):
            for half in range(2):
                pltpu.make_async_copy(
                    mh.at[idxall.at[half, t]],
                    rv.at[p].at[pl.ds(half * 128, 128)], sem).wait()

    start_gathers(0, 0)

    @pl.loop(0, BPW)
    def _batch(t):
        b = b0 + t
        p = lax.rem(t, 2)
        drain_gathers(t, p)

        @pl.when(t + 1 < BPW)
        def _prefetch():
            start_gathers(t + 1, 1 - p)

        lvec = [vl[t, pl.ds(16 * j, 16)] for j in range(4)]
        avec = [vab[t, pl.ds(16 * j, 16)] for j in range(4)]
        ovec = [vori[t, pl.ds(16 * j, 16)] for j in range(4)]

        @pl.loop(0, KP1 // 16)
        def _kc(kc):
            base = kc * 16
            for j in range(16):
                k = base + j
                wlk = [rows_l[p, k, pl.ds(16 * q, 16)] for q in range(4)]
                wak = [rows_ab[p, k, pl.ds(16 * q, 16)] for q in range(4)]
                wok = [rows_ori[p, k, pl.ds(16 * q, 16)] for q in range(4)]

                def pdot(wv, vv):
                    return (wv[0] * vv[0] + wv[1] * vv[1]
                            + wv[2] * vv[2] + wv[3] * vv[3])

                colj = jnp.full((16,), j, jnp.int32)
                pds = (pdot(wlk, avec),
                       pdot(wak, lvec),
                       pdot(wlk, ovec),
                       pdot(wok, lvec),
                       pdot(wok, avec),
                       pdot(wak, ovec))
                for i in range(6):
                    plsc.store_scatter(redbuf.at[i], (iota16, colj), pds[i])
            for i in range(6):
                acc = redbuf[i, 0, pl.ds(0, 16)]
                for r in range(1, 16):
                    acc = acc + redbuf[i, r, pl.ds(0, 16)]
                out6[i, pl.ds(base, 16)] = acc

        ho = []
        for i, oref in enumerate((o0, o1, o2, o3, o4, o5)):
            ho.append(pltpu.async_copy(out6.at[i], oref.at[b], sem2))
        for h in ho:
            h.wait()

    for tc in range(BPW // 16):
        for j in range(16):
            t = tc * 16 + j
            yi = ysp[pl.ds(t, 16)][0]

            @pl.loop(0, B // 16, init_carry=jnp.full((16,), -1, jnp.int32))
            def best(ci, acc):
                yv = y_all[pl.ds(ci * 16, 16)]
                jv = iota16 + ci * 16
                return jnp.maximum(acc, jnp.where(yv == yi, jv, -1))

            plsc.store_scatter(ljbuf, (iota16, jnp.full((16,), j, jnp.int32)),
                               best)
        mx = ljbuf[0, pl.ds(0, 16)]
        for r in range(1, 16):
            mx = jnp.maximum(mx, ljbuf[r, pl.ds(0, 16)])
        lastj[pl.ds(tc * 16, 16)] = mx

    rbufs = (rows_l, rows_ab, rows_ori)
    hs = []
    for i, fh in enumerate((l_h, ab_h, ori_h)):
        hs.append(pltpu.async_copy(fh.at[lastj],
                                   rbufs[i].at[0, pl.ds(BPW, BPW)], sem))
    for i, mh in enumerate((ml_h, mab_h, mori_h)):
        hs.append(pltpu.async_copy(mh.at[ys],
                                   rbufs[i].at[0, pl.ds(0, BPW)], sem))
    for h in hs:
        h.wait()

    def _pos(m, t):
        return [rbufs[m][0, t, pl.ds(16 * j, 16)] * MOM
                + rbufs[m][0, BPW + t, pl.ds(16 * j, 16)] * (1.0 - MOM)
                for j in range(4)]

    for m in range(3):
        for g in range(BPW // 16):
            for j in range(16):
                pv = _pos(m, g * 16 + j)
                sq = (pv[0] * pv[0] + pv[1] * pv[1]
                      + pv[2] * pv[2] + pv[3] * pv[3])
                plsc.store_scatter(redbuf.at[0],
                                   (iota16, jnp.full((16,), j, jnp.int32)), sq)
            ns = redbuf[0, 0, pl.ds(0, 16)]
            for r in range(1, 16):
                ns = ns + redbuf[0, r, pl.ds(0, 16)]
            bits = plsc.bitcast(ns, jnp.int32)
            bits = jnp.int32(0x5F3759DF) - (bits >> 1)
            r = plsc.bitcast(bits, jnp.float32)
            for _ in range(4):
                r = r * (1.5 - 0.5 * ns * r * r)
            for j in range(16):
                t = g * 16 + j
                pv = _pos(m, t)
                rj = r[j]
                for q in range(4):
                    rbufs[m][0, 2 * BPW + t, pl.ds(16 * q, 16)] = pv[q] * rj

    for m in range(3):
        pltpu.sync_copy(rbufs[m].at[0, pl.ds(2 * BPW, BPW)],
                        upd_o.at[m, pl.ds(b0, BPW)])


RPW = N // NW


@functools.partial(
    pl.kernel,
    out_type=tuple(_f32(N + B, D) for _ in range(3)),
    mesh=_mesh,
    compiler_params=pltpu.CompilerParams(
        needs_layout_passes=False, use_tc_tiling_on_sc=False),
    scratch_types=[
        pltpu.VMEM((8, 128), jnp.int32),
        pltpu.VMEM((128, D), jnp.float32),
        pltpu.VMEM((B,), jnp.int32),
        pltpu.VMEM((2, 625, D), jnp.float32),
        pltpu.SemaphoreType.DMA,
        pltpu.SemaphoreType.DMA,
        pltpu.SemaphoreType.DMA,
    ],
)
def _scatter_sc(y_h, upd_h, ml_h, mab_h, mori_h, nl_o, nab_o, nori_o,
                destidx, updchunk, yb, copybuf, sem, semr, semw):
    c = lax.axis_index("c")
    s = lax.axis_index("s")
    w = s * NC + c
    r0 = w * RPW
    iota16 = lax.iota(jnp.int32, 16)

    NCH = RPW // 625
    hw = {}
    for mi, (mh, no) in enumerate(
            ((ml_h, nl_o), (mab_h, nab_o), (mori_h, nori_o))):
        for ci in range(NCH):
            gi = mi * NCH + ci
            p = gi % 2
            if gi >= 2:
                hw[gi - 2].wait()
            off = r0 + ci * 625
            pltpu.async_copy(mh.at[pl.ds(off, 625)], copybuf.at[p],
                             semr).wait()
            hw[gi] = pltpu.async_copy(copybuf.at[p], no.at[pl.ds(off, 625)],
                                      semw)
    hw[3 * NCH - 2].wait()
    hw[3 * NCH - 1].wait()

    pltpu.sync_copy(y_h, yb)
    for c8 in range(8):
        for j in range(8):
            i0 = c8 * 128 + j * 16
            yv = yb[pl.ds(i0, 16)]
            dv = jnp.where((yv >= r0) & (yv < r0 + RPW), yv, N + i0 + iota16)
            destidx[c8, pl.ds(j * 16, 16)] = dv

    for m, no in enumerate((nl_o, nab_o, nori_o)):
        for c8 in range(8):
            pltpu.sync_copy(upd_h.at[m, pl.ds(c8 * 128, 128)], updchunk)
            pltpu.async_copy(updchunk, no.at[destidx.at[c8]], sem).wait()


def kernel(l, ab, ori, y, idx, memory_l, memory_ab, memory_ori):
    y32 = y.astype(jnp.int32)
    idx32 = idx.astype(jnp.int32)
    o0, o1, o2, o3, o4, o5, upd = _nce_sc(
        l, ab, ori, y32, idx32, memory_l, memory_ab, memory_ori)
    nl, nab, nori = _scatter_sc(y32, upd, memory_l, memory_ab, memory_ori)
    return (o0[..., None], o1[..., None], o2[..., None], o3[..., None],
            o4[..., None], o5[..., None], nl[:N], nab[:N], nori[:N])


# speedup vs baseline: 3.3818x; 1.0028x over previous
import functools

import jax
import jax.numpy as jnp
from jax import lax
from jax.experimental import pallas as pl
from jax.experimental.pallas import tpu as pltpu
from jax.experimental.pallas import tpu_sc as plsc

B = 1024
KP1 = 256
D = 64
N = 100000
MOM = 0.5
NC = 2
NS = 16
NW = NC * NS
BPW = B // NW

_mesh = plsc.VectorSubcoreMesh(
    core_axis_name="c", subcore_axis_name="s", num_cores=NC, num_subcores=NS
)


def _f32(*s):
    return jax.ShapeDtypeStruct(s, jnp.float32)


@functools.partial(
    pl.kernel,
    out_type=tuple(_f32(B, KP1) for _ in range(6)) + (_f32(3, B, D),),
    mesh=_mesh,
    compiler_params=pltpu.CompilerParams(
        needs_layout_passes=False, use_tc_tiling_on_sc=False),
    scratch_types=[
        pltpu.VMEM((2, BPW, 128), jnp.int32),
        pltpu.VMEM((2, KP1, D), jnp.float32),
        pltpu.VMEM((2, KP1, D), jnp.float32),
        pltpu.VMEM((2, KP1, D), jnp.float32),
        pltpu.VMEM((BPW, D), jnp.float32),
        pltpu.VMEM((BPW, D), jnp.float32),
        pltpu.VMEM((BPW, D), jnp.float32),
        pltpu.VMEM((B,), jnp.int32),
        pltpu.VMEM((BPW,), jnp.int32),
        pltpu.VMEM((BPW + 16,), jnp.int32),
        pltpu.VMEM((BPW,), jnp.int32),
        pltpu.VMEM((2, 6, KP1), jnp.float32),
        pltpu.VMEM((6, 16, 16), jnp.float32),
        pltpu.VMEM((16, 16), jnp.int32),
        pltpu.SemaphoreType.DMA,
        pltpu.SemaphoreType.DMA,
    ],
)
def _nce_sc(l_h, ab_h, ori_h, y_h, idx_h, ml_h, mab_h, mori_h,
            o0, o1, o2, o3, o4, o5, upd_o,
            idxall, rows_l, rows_ab, rows_ori, vl, vab, vori,
            y_all, ys, ysp, lastj, out6, redbuf, ljbuf,
            sem, sem2):
    c = lax.axis_index("c")
    s = lax.axis_index("s")
    w = s * NC + c
    b0 = w * BPW
    iota16 = lax.iota(jnp.int32, 16)

    pltpu.sync_copy(y_h, y_all)
    pltpu.sync_copy(y_h.at[pl.ds(b0, BPW)], ys)
    pltpu.sync_copy(y_h.at[pl.ds(b0, BPW)], ysp.at[pl.ds(0, BPW)])
    pltpu.sync_copy(l_h.at[pl.ds(b0, BPW)], vl)
    pltpu.sync_copy(ab_h.at[pl.ds(b0, BPW)], vab)
    pltpu.sync_copy(ori_h.at[pl.ds(b0, BPW)], vori)
    pltpu.sync_copy(idx_h.at[pl.ds(b0, BPW), pl.ds(0, 128)], idxall.at[0])
    pltpu.sync_copy(idx_h.at[pl.ds(b0, BPW), pl.ds(128, 128)], idxall.at[1])
    for t in range(BPW):
        yb = ysp[pl.ds(t, 16)][0]
        v0 = idxall[0, t, pl.ds(0, 16)]
        idxall[0, t, pl.ds(0, 16)] = jnp.where(iota16 == 0, yb, v0)

    def start_gathers(t, p):
        for mh, rv in ((ml_h, rows_l), (mab_h, rows_ab), (mori_h, rows_ori)):
            for half in range(2):
                pltpu.async_copy(
                    mh.at[idxall.at[half, t]],
                    rv.at[p].at[pl.ds(half * 128, 128)], sem)

    def drain_gathers(t, p):
        for mh, rv in ((ml_h, rows_l), (mab_h, rows_ab), (mori_h, rows_ori)):
            for half in range(2):
                pltpu.make_async_copy(
                    mh.at[idxall.at[half, t]],
                    rv.at[p].at[pl.ds(half * 128, 128)], sem).wait()

    start_gathers(0, 0)

    @pl.loop(0, BPW)
    def _batch(t):
        b = b0 + t
        p = lax.rem(t, 2)
        drain_gathers(t, p)

        @pl.when(t + 1 < BPW)
        def _prefetch():
            start_gathers(t + 1, 1 - p)

        @pl.when(t >= 2)
        def _drain_prev():
            for i, oref in enumerate((o0, o1, o2, o3, o4, o5)):
                pltpu.make_async_copy(out6.at[p, i], oref.at[b], sem2).wait()

        lvec = [vl[t, pl.ds(16 * j, 16)] for j in range(4)]
        avec = [vab[t, pl.ds(16 * j, 16)] for j in range(4)]
        ovec = [vori[t, pl.ds(16 * j, 16)] for j in range(4)]

        @pl.loop(0, KP1 // 16)
        def _kc(kc):
            base = kc * 16
            for j in range(16):
                k = base + j
                wlk = [rows_l[p, k, pl.ds(16 * q, 16)] for q in range(4)]
                wak = [rows_ab[p, k, pl.ds(16 * q, 16)] for q in range(4)]
                wok = [rows_ori[p, k, pl.ds(16 * q, 16)] for q in range(4)]

                def pdot(wv, vv):
                    return (wv[0] * vv[0] + wv[1] * vv[1]
                            + wv[2] * vv[2] + wv[3] * vv[3])

                colj = jnp.full((16,), j, jnp.int32)
                pds = (pdot(wlk, avec),
                       pdot(wak, lvec),
                       pdot(wlk, ovec),
                       pdot(wok, lvec),
                       pdot(wok, avec),
                       pdot(wak, ovec))
                for i in range(6):
                    plsc.store_scatter(redbuf.at[i], (iota16, colj), pds[i])
            for i in range(6):
                acc = redbuf[i, 0, pl.ds(0, 16)]
                for r in range(1, 16):
                    acc = acc + redbuf[i, r, pl.ds(0, 16)]
                out6[p, i, pl.ds(base, 16)] = acc

        for i, oref in enumerate((o0, o1, o2, o3, o4, o5)):
            pltpu.async_copy(out6.at[p, i], oref.at[b], sem2)

    for pp in range(2):
        for i, oref in enumerate((o0, o1, o2, o3, o4, o5)):
            pltpu.make_async_copy(out6.at[pp, i], oref.at[b0], sem2).wait()

    for tc in range(BPW // 16):
        for j in range(16):
            t = tc * 16 + j
            yi = ysp[pl.ds(t, 16)][0]

            @pl.loop(0, B // 16, init_carry=jnp.full((16,), -1, jnp.int32))
            def best(ci, acc):
                yv = y_all[pl.ds(ci * 16, 16)]
                jv = iota16 + ci * 16
                return jnp.maximum(acc, jnp.where(yv == yi, jv, -1))

            plsc.store_scatter(ljbuf, (iota16, jnp.full((16,), j, jnp.int32)),
                               best)
        mx = ljbuf[0, pl.ds(0, 16)]
        for r in range(1, 16):
            mx = jnp.maximum(mx, ljbuf[r, pl.ds(0, 16)])
        lastj[pl.ds(tc * 16, 16)] = mx

    rbufs = (rows_l, rows_ab, rows_ori)
    hs = []
    for i, fh in enumerate((l_h, ab_h, ori_h)):
        hs.append(pltpu.async_copy(fh.at[lastj],
                                   rbufs[i].at[0, pl.ds(BPW, BPW)], sem))
    for i, mh in enumerate((ml_h, mab_h, mori_h)):
        hs.append(pltpu.async_copy(mh.at[ys],
                                   rbufs[i].at[0, pl.ds(0, BPW)], sem))
    for h in hs:
        h.wait()

    def _pos(m, t):
        return [rbufs[m][0, t, pl.ds(16 * j, 16)] * MOM
                + rbufs[m][0, BPW + t, pl.ds(16 * j, 16)] * (1.0 - MOM)
                for j in range(4)]

    for m in range(3):
        for g in range(BPW // 16):
            for j in range(16):
                pv = _pos(m, g * 16 + j)
                sq = (pv[0] * pv[0] + pv[1] * pv[1]
                      + pv[2] * pv[2] + pv[3] * pv[3])
                plsc.store_scatter(redbuf.at[0],
                                   (iota16, jnp.full((16,), j, jnp.int32)), sq)
            ns = redbuf[0, 0, pl.ds(0, 16)]
            for r in range(1, 16):
                ns = ns + redbuf[0, r, pl.ds(0, 16)]
            bits = plsc.bitcast(ns, jnp.int32)
            bits = jnp.int32(0x5F3759DF) - (bits >> 1)
            r = plsc.bitcast(bits, jnp.float32)
            for _ in range(4):
                r = r * (1.5 - 0.5 * ns * r * r)
            for j in range(16):
                t = g * 16 + j
                pv = _pos(m, t)
                rj = r[j]
                for q in range(4):
                    rbufs[m][0, 2 * BPW + t, pl.ds(16 * q, 16)] = pv[q] * rj

    for m in range(3):
        pltpu.sync_copy(rbufs[m].at[0, pl.ds(2 * BPW, BPW)],
                        upd_o.at[m, pl.ds(b0, BPW)])


RPW = N // NW


@functools.partial(
    pl.kernel,
    out_type=tuple(_f32(N + B, D) for _ in range(3)),
    mesh=_mesh,
    compiler_params=pltpu.CompilerParams(
        needs_layout_passes=False, use_tc_tiling_on_sc=False),
    scratch_types=[
        pltpu.VMEM((8, 128), jnp.int32),
        pltpu.VMEM((128, D), jnp.float32),
        pltpu.VMEM((B,), jnp.int32),
        pltpu.VMEM((2, 625, D), jnp.float32),
        pltpu.SemaphoreType.DMA,
        pltpu.SemaphoreType.DMA,
        pltpu.SemaphoreType.DMA,
    ],
)
def _scatter_sc(y_h, upd_h, ml_h, mab_h, mori_h, nl_o, nab_o, nori_o,
                destidx, updchunk, yb, copybuf, sem, semr, semw):
    c = lax.axis_index("c")
    s = lax.axis_index("s")
    w = s * NC + c
    r0 = w * RPW
    iota16 = lax.iota(jnp.int32, 16)

    NCH = RPW // 625
    hw = {}
    for mi, (mh, no) in enumerate(
            ((ml_h, nl_o), (mab_h, nab_o), (mori_h, nori_o))):
        for ci in range(NCH):
            gi = mi * NCH + ci
            p = gi % 2
            if gi >= 2:
                hw[gi - 2].wait()
            off = r0 + ci * 625
            pltpu.async_copy(mh.at[pl.ds(off, 625)], copybuf.at[p],
                             semr).wait()
            hw[gi] = pltpu.async_copy(copybuf.at[p], no.at[pl.ds(off, 625)],
                                      semw)
    hw[3 * NCH - 2].wait()
    hw[3 * NCH - 1].wait()

    pltpu.sync_copy(y_h, yb)
    for c8 in range(8):
        for j in range(8):
            i0 = c8 * 128 + j * 16
            yv = yb[pl.ds(i0, 16)]
            dv = jnp.where((yv >= r0) & (yv < r0 + RPW), yv, N + i0 + iota16)
            destidx[c8, pl.ds(j * 16, 16)] = dv

    for m, no in enumerate((nl_o, nab_o, nori_o)):
        for c8 in range(8):
            pltpu.sync_copy(upd_h.at[m, pl.ds(c8 * 128, 128)], updchunk)
            pltpu.async_copy(updchunk, no.at[destidx.at[c8]], sem).wait()


def kernel(l, ab, ori, y, idx, memory_l, memory_ab, memory_ori):
    y32 = y.astype(jnp.int32)
    idx32 = idx.astype(jnp.int32)
    o0, o1, o2, o3, o4, o5, upd = _nce_sc(
        l, ab, ori, y32, idx32, memory_l, memory_ab, memory_ori)
    nl, nab, nori = _scatter_sc(y32, upd, memory_l, memory_ab, memory_ori)
    return (o0[..., None], o1[..., None], o2[..., None], o3[..., None],
            o4[..., None], o5[..., None], nl[:N], nab[:N], nori[:N])
